# baseline (speedup 1.0000x reference)
import functools

import jax
import jax.numpy as jnp
import numpy as np
from jax import lax
from jax.experimental import pallas as pl
from jax.experimental.pallas import tpu as pltpu
from jax.experimental.pallas import tpu_sc as plsc

NC = 2
NS = 16
NW = NC * NS
CHUNK = 128
K = 4
NSETS = 3
AHEAD = 2

_MESH = plsc.VectorSubcoreMesh(
    core_axis_name="c", subcore_axis_name="s", num_cores=NC, num_subcores=NS)
_SC_PARAMS = pltpu.CompilerParams(use_tc_tiling_on_sc=False)
_SC_VPARAMS = pltpu.CompilerParams(
    use_tc_tiling_on_sc=False, needs_layout_passes=False)


def _prologue(sid, wid, sl, zsrc, acc, idx_pairs):
  for i in range(sl // CHUNK):
    pltpu.sync_copy(zsrc, acc.at[pl.ds(sid * sl + i * CHUNK, CHUNK)])
  for hbm, vmem in idx_pairs:
    pltpu.sync_copy(hbm, vmem)
  plsc.subcore_barrier()


def _epilogue(cid, sid, sl, acc, out_hbm):
  plsc.subcore_barrier()
  pltpu.sync_copy(acc.at[pl.ds(sid * sl, sl)],
                  out_hbm.at[cid, pl.ds(sid * sl, sl)])


def _deg_kernel(npad, cpw):
  sl = npad // NS

  @functools.partial(
      pl.kernel,
      out_type=jax.ShapeDtypeStruct((NC, npad), jnp.float32),
      mesh=_MESH,
      compiler_params=_SC_PARAMS,
      scratch_types=[
          pltpu.VMEM((cpw, CHUNK), jnp.int32),
          pltpu.VMEM((CHUNK,), jnp.float32),
          pltpu.VMEM((CHUNK,), jnp.float32),
          pltpu.VMEM_SHARED((npad,), jnp.float32),
          pltpu.SemaphoreType.DMA,
      ],
  )
  def k(slab_hbm, out_hbm, cidx, ones, zeros, acc, ssem):
    cid = lax.axis_index("c")
    sid = lax.axis_index("s")
    wid = sid * NC + cid

    def fill(i, _):
      ones[pl.ds(i * 16, 16)] = jnp.ones((16,), jnp.float32)
      zeros[pl.ds(i * 16, 16)] = jnp.zeros((16,), jnp.float32)
      return 0

    lax.fori_loop(0, CHUNK // 16, fill, 0)
    _prologue(sid, wid, sl, zeros, acc, [(slab_hbm.at[1, wid], cidx)])

    pend = []
    for g in range(0, cpw, K):
      if len(pend) > K:
        for _ in range(K):
          pend.pop(0).wait()
      for j in range(g, g + K):
        pend.append(
            pltpu.async_copy(ones, acc.at[cidx.at[j]], ssem, add=True))
    for d in pend:
      d.wait()

    _epilogue(cid, sid, sl, acc, out_hbm)

  return k


def _agg2d_kernel(npad, cpw, h):
  sl = npad // NS
  ng = cpw // K
  assert cpw % K == 0
  per_row = h // 16

  @functools.partial(
      pl.kernel,
      out_type=jax.ShapeDtypeStruct((NC, npad, h), jnp.float32),
      mesh=_MESH,
      compiler_params=_SC_PARAMS,
      scratch_types=[
          pltpu.VMEM((cpw, CHUNK), jnp.int32),
          pltpu.VMEM((cpw, CHUNK), jnp.int32),
          pltpu.VMEM((NSETS, K, CHUNK, h), jnp.float32),
          pltpu.VMEM_SHARED((npad, h), jnp.float32),
          pltpu.VMEM_SHARED((npad, h), jnp.float32),
          pltpu.SemaphoreType.DMA,
          pltpu.SemaphoreType.DMA,
      ],
  )
  def k(u_hbm, slab_hbm, out_hbm, ridx, cidx, gbuf, acc, ush, gsem, ssem):
    cid = lax.axis_index("c")
    sid = lax.axis_index("s")
    wid = sid * NC + cid
    n = u_hbm.shape[0]

    def zfill(i, _):
      gbuf[0, 0, i // per_row, pl.ds((i % per_row) * 16, 16)] = jnp.zeros(
          (16,), jnp.float32)
      return 0

    lax.fori_loop(0, CHUNK * per_row, zfill, 0)
    usl = n // NS
    pltpu.sync_copy(u_hbm.at[pl.ds(sid * usl, usl)],
                    ush.at[pl.ds(sid * usl, usl)])
    _prologue(sid, wid, sl, gbuf.at[0, 0], acc,
              [(slab_hbm.at[0, wid], ridx), (slab_hbm.at[1, wid], cidx)])

    def fire_gathers(g):
      b = g % NSETS
      return [
          pltpu.async_copy(ush.at[ridx.at[g * K + i]], gbuf.at[b, i], gsem)
          for i in range(K)
      ]

    def fire_scatters(g):
      b = g % NSETS
      return [
          pltpu.async_copy(gbuf.at[b, i], acc.at[cidx.at[g * K + i]], ssem,
                           add=True)
          for i in range(K)
      ]

    pend_g = {}
    pend_s = {}
    for f in range(min(AHEAD, ng)):
      pend_g[f] = fire_gathers(f)
    for g in range(ng):
      f = g + AHEAD
      if f < ng:
        for d in pend_s.pop(f - NSETS, []):
          d.wait()
        pend_g[f] = fire_gathers(f)
      for d in pend_g.pop(g):
        d.wait()
      pend_s[g] = fire_scatters(g)
    for ds in pend_s.values():
      for d in ds:
        d.wait()

    _epilogue(cid, sid, sl, acc, out_hbm)

  return k


def _agg1d_kernel(npad, cpw, n):
  sl = npad // NS
  nbuf = 4

  @functools.partial(
      pl.kernel,
      out_type=jax.ShapeDtypeStruct((NC, npad), jnp.float32),
      mesh=_MESH,
      compiler_params=_SC_VPARAMS,
      scratch_types=[
          pltpu.VMEM((cpw, CHUNK), jnp.int32),
          pltpu.VMEM((cpw, CHUNK), jnp.int32),
          pltpu.VMEM((n,), jnp.float32),
          pltpu.VMEM((nbuf, CHUNK), jnp.float32),
          pltpu.VMEM_SHARED((npad,), jnp.float32),
          pltpu.SemaphoreType.DMA,
      ],
  )
  def k(u_hbm, slab_hbm, out_hbm, ridx, cidx, u2t, gbuf, acc, ssem):
    cid = lax.axis_index("c")
    sid = lax.axis_index("s")
    wid = sid * NC + cid

    def zfill(i, _):
      gbuf[0, pl.ds(i * 16, 16)] = jnp.zeros((16,), jnp.float32)
      return 0

    lax.fori_loop(0, CHUNK // 16, zfill, 0)
    pltpu.sync_copy(u_hbm, u2t)
    _prologue(sid, wid, sl, gbuf.at[0], acc,
              [(slab_hbm.at[0, wid], ridx), (slab_hbm.at[1, wid], cidx)])

    pend = {}
    for j in range(cpw):
      b = j % nbuf
      for d in pend.pop(j - nbuf, []):
        d.wait()
      for t in range(CHUNK // 16):
        idxv = ridx[j, pl.ds(t * 16, 16)]
        gbuf[b, pl.ds(t * 16, 16)] = plsc.load_gather(u2t, [idxv])
      pend[j] = [
          pltpu.async_copy(gbuf.at[b], acc.at[cidx.at[j]], ssem, add=True)
      ]
    for ds in pend.values():
      for d in ds:
        d.wait()

    _epilogue(cid, sid, sl, acc, out_hbm)

  return k


def _expand_mat(p, h):
  r = lax.broadcasted_iota(jnp.int32, (p, p * h), 0)
  c = lax.broadcasted_iota(jnp.int32, (p, p * h), 1)
  return jnp.where(r == c // h, 1.0, 0.0).astype(jnp.float32)


def _blockdiag(w, p):
  d, h = w.shape
  t = jnp.tile(w, (p, p))
  r = lax.broadcasted_iota(jnp.int32, (p * d, p * h), 0)
  c = lax.broadcasted_iota(jnp.int32, (p * d, p * h), 1)
  return jnp.where(r // d == c // h, t, 0.0)


def _tc_a(x4, w1, degp4, n4, q4, p):
  def body(x_ref, w1_ref, degp_ref, u_ref, dinvq_ref):
    h = w1_ref.shape[1]
    w14 = _blockdiag(w1_ref[...], p)
    xw4 = jnp.dot(x_ref[...], w14,
                  preferred_element_type=jnp.float32)
    degq = degp_ref[0] + degp_ref[1] + 1.0
    dinvq = lax.rsqrt(degq)
    dinv4 = jnp.dot(dinvq[:n4, :], _expand_mat(p, h),
                    preferred_element_type=jnp.float32)
    u_ref[...] = xw4 * dinv4
    dinvq_ref[...] = dinvq

  return pl.pallas_call(
      body,
      out_shape=(jax.ShapeDtypeStruct((n4, 128), jnp.float32),
                 jax.ShapeDtypeStruct((q4, p), jnp.float32)),
  )(x4, w1, degp4)


def _tc_b(accp4, u4, dinvq, b1, w2, n4, p):
  def body(accp_ref, u_ref, dinvq_ref, b1_ref, w2_ref, u2_ref):
    h = b1_ref.shape[0]
    dq = dinvq_ref[:n4, :]
    dinv4 = jnp.dot(dq, _expand_mat(p, h),
                    preferred_element_type=jnp.float32)
    acc4 = accp_ref[0, :n4, :] + accp_ref[1, :n4, :] + u_ref[...]
    b1_4 = jnp.tile(b1_ref[...], p)
    out1 = acc4 * dinv4 + b1_4[None, :]
    hid = jnp.maximum(out1, 0.0)
    w24 = _blockdiag(w2_ref[...], p)
    hw2q = jnp.dot(hid, w24,
                   preferred_element_type=jnp.float32)
    u2_ref[...] = hw2q * dq

  return pl.pallas_call(
      body,
      out_shape=jax.ShapeDtypeStruct((n4, p), jnp.float32),
  )(accp4, u4, dinvq, b1, w2)


def _tc_c(acc2q, u2q, dinvq, b2, n4):
  def body(acc2_ref, u2_ref, dinvq_ref, b2_ref, out_ref):
    q = acc2_ref[0, :n4, :] + acc2_ref[1, :n4, :] + u2_ref[...]
    out_ref[...] = jax.nn.sigmoid(q * dinvq_ref[:n4, :] + b2_ref[0])

  return pl.pallas_call(
      body,
      out_shape=jax.ShapeDtypeStruct(u2q.shape, jnp.float32),
  )(acc2q, u2q, dinvq, b2)


def kernel(x, edge_index, W1, b1, W2, b2):
  n = x.shape[0]
  d = x.shape[1]
  e = edge_index.shape[1]
  h = W1.shape[1]

  npad = ((n + NS * CHUNK - 1) // (NS * CHUNK)) * (NS * CHUNK)
  gsz = NW * CHUNK * K
  cpw = K * ((e + gsz - 1) // gsz)
  ep = NW * cpw * CHUNK

  npr = np.arange(ep - e, dtype=np.int32)
  pad2 = jnp.asarray(np.stack([npr % n, n + npr % (npad - n)]))
  slab = jnp.concatenate([edge_index, pad2], axis=1).reshape(
      2, NW, cpw, CHUNK)

  p = 128 // h
  n4 = n // p
  q4 = npad // p

  degp = _deg_kernel(npad, cpw)(slab)
  u4, dinvq = _tc_a(x.reshape(n4, p * d), W1,
                    degp.reshape(NC, q4, p), n4, q4, p)
  accp = _agg2d_kernel(npad, cpw, h)(u4.reshape(n, h), slab)
  u2q = _tc_b(accp.reshape(NC, q4, 128), u4, dinvq, b1, W2, n4, p)
  acc2p = _agg1d_kernel(npad, cpw, n)(u2q.reshape(n), slab)
  outq = _tc_c(acc2p.reshape(NC, q4, p), u2q, dinvq, b2, n4)
  return outq.reshape(n)


# speedup vs baseline: 1.2208x; 1.2208x over previous
import functools

import jax
import jax.numpy as jnp
import numpy as np
from jax import lax
from jax.experimental import pallas as pl
from jax.experimental.pallas import tpu as pltpu
from jax.experimental.pallas import tpu_sc as plsc

NC = 2
NS = 16
NW = NC * NS
CHUNK = 128
K = 4
NSETS = 5
AHEAD = 3

_MESH = plsc.VectorSubcoreMesh(
    core_axis_name="c", subcore_axis_name="s", num_cores=NC, num_subcores=NS)
_SC_PARAMS = pltpu.CompilerParams(use_tc_tiling_on_sc=False)
_SC_VPARAMS = pltpu.CompilerParams(
    use_tc_tiling_on_sc=False, needs_layout_passes=False)


def _prologue(sid, wid, sl, zsrc, acc, idx_pairs):
  for i in range(sl // CHUNK):
    pltpu.sync_copy(zsrc, acc.at[pl.ds(sid * sl + i * CHUNK, CHUNK)])
  for hbm, vmem in idx_pairs:
    pltpu.sync_copy(hbm, vmem)
  plsc.subcore_barrier()


def _epilogue(cid, sid, sl, acc, out_hbm):
  plsc.subcore_barrier()
  pltpu.sync_copy(acc.at[pl.ds(sid * sl, sl)],
                  out_hbm.at[cid, pl.ds(sid * sl, sl)])


def _deg_kernel(npad, cpw):
  sl = npad // NS

  @functools.partial(
      pl.kernel,
      out_type=jax.ShapeDtypeStruct((NC, npad), jnp.float32),
      mesh=_MESH,
      compiler_params=_SC_PARAMS,
      scratch_types=[
          pltpu.VMEM((cpw, CHUNK), jnp.int32),
          pltpu.VMEM((CHUNK,), jnp.float32),
          pltpu.VMEM((CHUNK,), jnp.float32),
          pltpu.VMEM_SHARED((npad,), jnp.float32),
          pltpu.SemaphoreType.DMA,
      ],
  )
  def k(slab_hbm, out_hbm, cidx, ones, zeros, acc, ssem):
    cid = lax.axis_index("c")
    sid = lax.axis_index("s")
    wid = sid * NC + cid

    def fill(i, _):
      ones[pl.ds(i * 16, 16)] = jnp.ones((16,), jnp.float32)
      zeros[pl.ds(i * 16, 16)] = jnp.zeros((16,), jnp.float32)
      return 0

    lax.fori_loop(0, CHUNK // 16, fill, 0)
    _prologue(sid, wid, sl, zeros, acc, [(slab_hbm.at[1, wid], cidx)])

    pend = []
    for g in range(0, cpw, K):
      if len(pend) > K:
        for _ in range(K):
          pend.pop(0).wait()
      for j in range(g, g + K):
        pend.append(
            pltpu.async_copy(ones, acc.at[cidx.at[j]], ssem, add=True))
    for d in pend:
      d.wait()

    _epilogue(cid, sid, sl, acc, out_hbm)

  return k


def _agg2d_kernel(npad, cpw, h):
  sl = npad // NS
  ng = cpw // K
  assert cpw % K == 0
  per_row = h // 16

  @functools.partial(
      pl.kernel,
      out_type=jax.ShapeDtypeStruct((NC, npad, h), jnp.float32),
      mesh=_MESH,
      compiler_params=_SC_PARAMS,
      scratch_types=[
          pltpu.VMEM((cpw, CHUNK), jnp.int32),
          pltpu.VMEM((cpw, CHUNK), jnp.int32),
          pltpu.VMEM((NSETS, K, CHUNK, h), jnp.float32),
          pltpu.VMEM_SHARED((npad, h), jnp.float32),
          pltpu.SemaphoreType.DMA,
          pltpu.SemaphoreType.DMA,
      ],
  )
  def k(u_hbm, slab_hbm, out_hbm, ridx, cidx, gbuf, acc, gsem, ssem):
    cid = lax.axis_index("c")
    sid = lax.axis_index("s")
    wid = sid * NC + cid

    def zfill(i, _):
      gbuf[0, 0, i // per_row, pl.ds((i % per_row) * 16, 16)] = jnp.zeros(
          (16,), jnp.float32)
      return 0

    lax.fori_loop(0, CHUNK * per_row, zfill, 0)
    _prologue(sid, wid, sl, gbuf.at[0, 0], acc,
              [(slab_hbm.at[0, wid], ridx), (slab_hbm.at[1, wid], cidx)])

    def fire_gathers(g):
      b = g % NSETS
      return [
          pltpu.async_copy(u_hbm.at[ridx.at[g * K + i]], gbuf.at[b, i], gsem)
          for i in range(K)
      ]

    def fire_scatters(g):
      b = g % NSETS
      return [
          pltpu.async_copy(gbuf.at[b, i], acc.at[cidx.at[g * K + i]], ssem,
                           add=True)
          for i in range(K)
      ]

    pend_g = {}
    pend_s = {}
    for f in range(min(AHEAD, ng)):
      pend_g[f] = fire_gathers(f)
    for g in range(ng):
      f = g + AHEAD
      if f < ng:
        for d in pend_s.pop(f - NSETS, []):
          d.wait()
        pend_g[f] = fire_gathers(f)
      for d in pend_g.pop(g):
        d.wait()
      pend_s[g] = fire_scatters(g)
    for ds in pend_s.values():
      for d in ds:
        d.wait()

    _epilogue(cid, sid, sl, acc, out_hbm)

  return k


def _agg1d_kernel(npad, cpw):
  sl = npad // NS
  nbuf = 4
  urows = npad // 4 // NS

  @functools.partial(
      pl.kernel,
      out_type=jax.ShapeDtypeStruct((NC, npad), jnp.float32),
      mesh=_MESH,
      compiler_params=_SC_VPARAMS,
      scratch_types=[
          pltpu.VMEM((cpw, CHUNK), jnp.int32),
          pltpu.VMEM((cpw, CHUNK), jnp.int32),
          pltpu.VMEM((npad // NS // 4, 128), jnp.float32),
          pltpu.VMEM((npad // NS // CHUNK, CHUNK), jnp.int32),
          pltpu.VMEM((npad // NS,), jnp.float32),
          pltpu.VMEM((npad,), jnp.float32),
          pltpu.VMEM((nbuf, CHUNK), jnp.float32),
          pltpu.VMEM_SHARED((npad,), jnp.float32),
          pltpu.VMEM_SHARED((npad,), jnp.float32),
          pltpu.SemaphoreType.DMA,
      ],
  )
  def k(u_hbm, slab_hbm, out_hbm, ridx, cidx, urep, iot, uown, u2t, gbuf,
        acc, u2sh, ssem):
    cid = lax.axis_index("c")
    sid = lax.axis_index("s")
    wid = sid * NC + cid

    def zfill(i, _):
      gbuf[0, pl.ds(i * 16, 16)] = jnp.zeros((16,), jnp.float32)
      return 0

    lax.fori_loop(0, CHUNK // 16, zfill, 0)

    pltpu.sync_copy(u_hbm.at[pl.ds(sid * urows, urows)], urep)

    def compact(i, _):
      j16 = jnp.arange(16, dtype=jnp.int32) + i * 16
      uown[pl.ds(i * 16, 16)] = plsc.load_gather(
          urep, [j16 >> 2, (j16 & 3) << 5])
      iot[i // 8, pl.ds((i % 8) * 16, 16)] = j16 + sid * sl
      return 0

    lax.fori_loop(0, sl // 16, compact, 0)
    pltpu.sync_copy(uown, u2sh.at[pl.ds(sid * sl, sl)])
    _prologue(sid, wid, sl, gbuf.at[0], acc,
              [(slab_hbm.at[0, wid], ridx), (slab_hbm.at[1, wid], cidx)])
    pltpu.sync_copy(u2sh, u2t)

    pend = {}
    for j in range(cpw):
      b = j % nbuf
      for d in pend.pop(j - nbuf, []):
        d.wait()
      for t in range(CHUNK // 16):
        idxv = ridx[j, pl.ds(t * 16, 16)]
        gbuf[b, pl.ds(t * 16, 16)] = plsc.load_gather(u2t, [idxv])
      pend[j] = [
          pltpu.async_copy(gbuf.at[b], acc.at[cidx.at[j]], ssem, add=True)
      ]
    @pl.when(cid == 0)
    def _():
      for c in range(sl // CHUNK):
        pltpu.async_copy(uown.at[pl.ds(c * CHUNK, CHUNK)],
                         acc.at[iot.at[c]], ssem, add=True).wait()

    for ds in pend.values():
      for d in ds:
        d.wait()

    _epilogue(cid, sid, sl, acc, out_hbm)

  return k


def _expand_mat(p, h):
  r = lax.broadcasted_iota(jnp.int32, (p, p * h), 0)
  c = lax.broadcasted_iota(jnp.int32, (p, p * h), 1)
  return jnp.where(r == c // h, 1.0, 0.0).astype(jnp.float32)


def _blockdiag(w, p):
  d, h = w.shape
  t = jnp.tile(w, (p, p))
  r = lax.broadcasted_iota(jnp.int32, (p * d, p * h), 0)
  c = lax.broadcasted_iota(jnp.int32, (p * d, p * h), 1)
  return jnp.where(r // d == c // h, t, 0.0)


def _tc_a(x4, w1, degp4, n4, q4, p):
  def body(x_ref, w1_ref, degp_ref, u_ref, dinvq_ref):
    h = w1_ref.shape[1]
    w14 = _blockdiag(w1_ref[...], p)
    xw4 = jnp.dot(x_ref[...], w14,
                  preferred_element_type=jnp.float32)
    degq = degp_ref[0] + degp_ref[1] + 1.0
    dinvq = lax.rsqrt(degq)
    dinv4 = jnp.dot(dinvq[:n4, :], _expand_mat(p, h),
                    preferred_element_type=jnp.float32)
    u_ref[...] = xw4 * dinv4
    dinvq_ref[...] = dinvq

  return pl.pallas_call(
      body,
      out_shape=(jax.ShapeDtypeStruct((n4, 128), jnp.float32),
                 jax.ShapeDtypeStruct((q4, p), jnp.float32)),
  )(x4, w1, degp4)


def _tc_b(accp4, u4, dinvq, b1, w2, n4, q4, p):
  def body(accp_ref, u_ref, dinvq_ref, b1_ref, w2_ref, u2_ref):
    h = b1_ref.shape[0]
    dinv4 = jnp.dot(dinvq_ref[:n4, :], _expand_mat(p, h),
                    preferred_element_type=jnp.float32)
    acc4 = accp_ref[0, :n4, :] + accp_ref[1, :n4, :] + u_ref[...]
    b1_4 = jnp.tile(b1_ref[...], p)
    out1 = acc4 * dinv4 + b1_4[None, :]
    hid = jnp.maximum(out1, 0.0)
    w24 = _blockdiag(w2_ref[...], p)
    hw2q = jnp.dot(hid, w24,
                   preferred_element_type=jnp.float32)
    hw2rep = jnp.dot(hw2q, _expand_mat(p, h),
                     preferred_element_type=jnp.float32)
    u2rep = hw2rep * dinv4
    u2_ref[...] = jnp.concatenate(
        [u2rep, jnp.zeros((q4 - n4, p * h), jnp.float32)], axis=0)

  return pl.pallas_call(
      body,
      out_shape=jax.ShapeDtypeStruct((q4, p * b1.shape[0]), jnp.float32),
  )(accp4, u4, dinvq, b1, w2)


def _tc_c(acc2lin, deglin, b2, nr):
  def body(acc2_ref, deg_ref, b2_ref, out_ref):
    deg = deg_ref[0] + deg_ref[1] + 1.0
    dinv = lax.rsqrt(deg)
    q = acc2_ref[0] + acc2_ref[1]
    out_ref[...] = jax.nn.sigmoid(q * dinv + b2_ref[0])

  return pl.pallas_call(
      body,
      out_shape=jax.ShapeDtypeStruct((nr, 128), jnp.float32),
  )(acc2lin, deglin, b2)


def kernel(x, edge_index, W1, b1, W2, b2):
  n = x.shape[0]
  d = x.shape[1]
  e = edge_index.shape[1]
  h = W1.shape[1]

  npad = ((n + NS * CHUNK - 1) // (NS * CHUNK)) * (NS * CHUNK)
  gsz = NW * CHUNK * K
  cpw = K * ((e + gsz - 1) // gsz)
  ep = NW * cpw * CHUNK

  npr = np.arange(ep - e, dtype=np.int32)
  pad2 = jnp.asarray(np.stack([npr % n, n + npr % (npad - n)]))
  slab = jnp.concatenate([edge_index, pad2], axis=1).reshape(
      2, NW, cpw, CHUNK)

  p = 128 // h
  n4 = n // p
  q4 = npad // p

  degp = _deg_kernel(npad, cpw)(slab)
  u4, dinvq = _tc_a(x.reshape(n4, p * d), W1,
                    degp.reshape(NC, q4, p), n4, q4, p)
  accp = _agg2d_kernel(npad, cpw, h)(u4.reshape(n, h), slab)
  u2rep = _tc_b(accp.reshape(NC, q4, 128), u4, dinvq, b1, W2, n4, q4, p)
  acc2p = _agg1d_kernel(npad, cpw)(u2rep, slab)
  outlin = _tc_c(acc2p.reshape(NC, npad // 128, 128),
                 degp.reshape(NC, npad // 128, 128), b2, npad // 128)
  return outlin.reshape(npad)[:n]


# speedup vs baseline: 1.2415x; 1.0169x over previous
import functools

import jax
import jax.numpy as jnp
import numpy as np
from jax import lax
from jax.experimental import pallas as pl
from jax.experimental.pallas import tpu as pltpu
from jax.experimental.pallas import tpu_sc as plsc

NC = 2
NS = 16
NW = NC * NS
CHUNK = 128
K = 4
NSETS = 5
AHEAD = 3

_MESH = plsc.VectorSubcoreMesh(
    core_axis_name="c", subcore_axis_name="s", num_cores=NC, num_subcores=NS)
_SC_PARAMS = pltpu.CompilerParams(use_tc_tiling_on_sc=False)
_SC_VPARAMS = pltpu.CompilerParams(
    use_tc_tiling_on_sc=False, needs_layout_passes=False)


def _prologue(sid, wid, sl, zsrc, acc, idx_pairs):
  for i in range(sl // CHUNK):
    pltpu.sync_copy(zsrc, acc.at[pl.ds(sid * sl + i * CHUNK, CHUNK)])
  for hbm, vmem in idx_pairs:
    pltpu.sync_copy(hbm, vmem)
  plsc.subcore_barrier()


def _epilogue(cid, sid, sl, acc, out_hbm):
  plsc.subcore_barrier()
  pltpu.sync_copy(acc.at[pl.ds(sid * sl, sl)],
                  out_hbm.at[cid, pl.ds(sid * sl, sl)])


def _deg_kernel(npad, cpw):
  sl = npad // NS

  @functools.partial(
      pl.kernel,
      out_type=jax.ShapeDtypeStruct((NC, npad), jnp.float32),
      mesh=_MESH,
      compiler_params=_SC_PARAMS,
      scratch_types=[
          pltpu.VMEM((cpw, CHUNK), jnp.int32),
          pltpu.VMEM((CHUNK,), jnp.float32),
          pltpu.VMEM((CHUNK,), jnp.float32),
          pltpu.VMEM_SHARED((npad,), jnp.float32),
          pltpu.SemaphoreType.DMA,
      ],
  )
  def k(slab_hbm, out_hbm, cidx, ones, zeros, acc, ssem):
    cid = lax.axis_index("c")
    sid = lax.axis_index("s")
    wid = sid * NC + cid

    def fill(i, _):
      ones[pl.ds(i * 16, 16)] = jnp.ones((16,), jnp.float32)
      zeros[pl.ds(i * 16, 16)] = jnp.zeros((16,), jnp.float32)
      return 0

    lax.fori_loop(0, CHUNK // 16, fill, 0)
    _prologue(sid, wid, sl, zeros, acc, [(slab_hbm.at[1, wid], cidx)])

    pend = []
    for g in range(0, cpw, K):
      if len(pend) > K:
        for _ in range(K):
          pend.pop(0).wait()
      for j in range(g, g + K):
        pend.append(
            pltpu.async_copy(ones, acc.at[cidx.at[j]], ssem, add=True))
    for d in pend:
      d.wait()

    _epilogue(cid, sid, sl, acc, out_hbm)

  return k


def _agg2d_kernel(npad, cpw, h):
  sl = npad // NS
  ng = cpw // K
  assert cpw % K == 0
  per_row = h // 16

  @functools.partial(
      pl.kernel,
      out_type=jax.ShapeDtypeStruct((NC, npad, h), jnp.float32),
      mesh=_MESH,
      compiler_params=_SC_PARAMS,
      scratch_types=[
          pltpu.VMEM((cpw, CHUNK), jnp.int32),
          pltpu.VMEM((cpw, CHUNK), jnp.int32),
          pltpu.VMEM((NSETS, K, CHUNK, h), jnp.float32),
          pltpu.VMEM_SHARED((npad, h), jnp.float32),
          pltpu.SemaphoreType.DMA,
          pltpu.SemaphoreType.DMA,
      ],
  )
  def k(u_hbm, slab_hbm, out_hbm, ridx, cidx, gbuf, acc, gsem, ssem):
    cid = lax.axis_index("c")
    sid = lax.axis_index("s")
    wid = sid * NC + cid

    def zfill(i, _):
      gbuf[0, 0, i // per_row, pl.ds((i % per_row) * 16, 16)] = jnp.zeros(
          (16,), jnp.float32)
      return 0

    lax.fori_loop(0, CHUNK * per_row, zfill, 0)
    _prologue(sid, wid, sl, gbuf.at[0, 0], acc,
              [(slab_hbm.at[0, wid], ridx), (slab_hbm.at[1, wid], cidx)])

    def fire_gathers(g):
      b = g % NSETS
      return [
          pltpu.async_copy(u_hbm.at[ridx.at[g * K + i]], gbuf.at[b, i], gsem)
          for i in range(K)
      ]

    def fire_scatters(g):
      b = g % NSETS
      return [
          pltpu.async_copy(gbuf.at[b, i], acc.at[cidx.at[g * K + i]], ssem,
                           add=True)
          for i in range(K)
      ]

    pend_g = {}
    pend_s = {}
    for f in range(min(AHEAD, ng)):
      pend_g[f] = fire_gathers(f)
    for g in range(ng):
      f = g + AHEAD
      if f < ng:
        for d in pend_s.pop(f - NSETS, []):
          d.wait()
        pend_g[f] = fire_gathers(f)
      for d in pend_g.pop(g):
        d.wait()
      pend_s[g] = fire_scatters(g)
    for ds in pend_s.values():
      for d in ds:
        d.wait()

    _epilogue(cid, sid, sl, acc, out_hbm)

  return k


def _agg1d_kernel(npad, cpw):
  sl = npad // NS
  nbuf = 8
  urows = npad // 4 // NS

  @functools.partial(
      pl.kernel,
      out_type=jax.ShapeDtypeStruct((NC, npad), jnp.float32),
      mesh=_MESH,
      compiler_params=_SC_VPARAMS,
      scratch_types=[
          pltpu.VMEM((cpw, CHUNK), jnp.int32),
          pltpu.VMEM((cpw, CHUNK), jnp.int32),
          pltpu.VMEM((npad // NS // 4, 128), jnp.float32),
          pltpu.VMEM((npad // NS // CHUNK, CHUNK), jnp.int32),
          pltpu.VMEM((npad // NS,), jnp.float32),
          pltpu.VMEM((npad,), jnp.float32),
          pltpu.VMEM((nbuf, CHUNK), jnp.float32),
          pltpu.VMEM_SHARED((npad,), jnp.float32),
          pltpu.VMEM_SHARED((npad,), jnp.float32),
          pltpu.SemaphoreType.DMA,
      ],
  )
  def k(u_hbm, slab_hbm, out_hbm, ridx, cidx, urep, iot, uown, u2t, gbuf,
        acc, u2sh, ssem):
    cid = lax.axis_index("c")
    sid = lax.axis_index("s")
    wid = sid * NC + cid

    def zfill(i, _):
      gbuf[0, pl.ds(i * 16, 16)] = jnp.zeros((16,), jnp.float32)
      return 0

    lax.fori_loop(0, CHUNK // 16, zfill, 0)

    pltpu.sync_copy(u_hbm.at[pl.ds(sid * urows, urows)], urep)

    def compact(i, _):
      j16 = jnp.arange(16, dtype=jnp.int32) + i * 16
      uown[pl.ds(i * 16, 16)] = plsc.load_gather(
          urep, [j16 >> 2, (j16 & 3) << 5])
      iot[i // 8, pl.ds((i % 8) * 16, 16)] = j16 + sid * sl
      return 0

    lax.fori_loop(0, sl // 16, compact, 0)
    pltpu.sync_copy(uown, u2sh.at[pl.ds(sid * sl, sl)])
    _prologue(sid, wid, sl, gbuf.at[0], acc,
              [(slab_hbm.at[0, wid], ridx), (slab_hbm.at[1, wid], cidx)])
    pltpu.sync_copy(u2sh, u2t)

    pend = {}
    for j in range(cpw):
      b = j % nbuf
      for d in pend.pop(j - nbuf, []):
        d.wait()
      for t in range(CHUNK // 16):
        idxv = ridx[j, pl.ds(t * 16, 16)]
        gbuf[b, pl.ds(t * 16, 16)] = plsc.load_gather(u2t, [idxv])
      pend[j] = [
          pltpu.async_copy(gbuf.at[b], acc.at[cidx.at[j]], ssem, add=True)
      ]
    @pl.when(cid == 0)
    def _():
      for c in range(sl // CHUNK):
        pltpu.async_copy(uown.at[pl.ds(c * CHUNK, CHUNK)],
                         acc.at[iot.at[c]], ssem, add=True).wait()

    for ds in pend.values():
      for d in ds:
        d.wait()

    _epilogue(cid, sid, sl, acc, out_hbm)

  return k


def _expand_mat(p, h):
  r = lax.broadcasted_iota(jnp.int32, (p, p * h), 0)
  c = lax.broadcasted_iota(jnp.int32, (p, p * h), 1)
  return jnp.where(r == c // h, 1.0, 0.0).astype(jnp.float32)


def _blockdiag(w, p):
  d, h = w.shape
  t = jnp.tile(w, (p, p))
  r = lax.broadcasted_iota(jnp.int32, (p * d, p * h), 0)
  c = lax.broadcasted_iota(jnp.int32, (p * d, p * h), 1)
  return jnp.where(r // d == c // h, t, 0.0)


def _tc_a(x4, w1, degp4, n4, q4, p):
  def body(x_ref, w1_ref, degp_ref, u_ref, dinvq_ref):
    h = w1_ref.shape[1]
    w14 = _blockdiag(w1_ref[...], p)
    xw4 = jnp.dot(x_ref[...], w14,
                  preferred_element_type=jnp.float32)
    degq = degp_ref[0] + degp_ref[1] + 1.0
    dinvq = lax.rsqrt(degq)
    dinv4 = jnp.dot(dinvq[:n4, :], _expand_mat(p, h),
                    preferred_element_type=jnp.float32)
    u_ref[...] = xw4 * dinv4
    dinvq_ref[...] = dinvq

  return pl.pallas_call(
      body,
      out_shape=(jax.ShapeDtypeStruct((n4, 128), jnp.float32),
                 jax.ShapeDtypeStruct((q4, p), jnp.float32)),
  )(x4, w1, degp4)


def _tc_b(accp4, u4, dinvq, b1, w2, n4, q4, p):
  def body(accp_ref, u_ref, dinvq_ref, b1_ref, w2_ref, u2_ref):
    h = b1_ref.shape[0]
    dinv4 = jnp.dot(dinvq_ref[:n4, :], _expand_mat(p, h),
                    preferred_element_type=jnp.float32)
    acc4 = accp_ref[0, :n4, :] + accp_ref[1, :n4, :] + u_ref[...]
    b1_4 = jnp.tile(b1_ref[...], p)
    out1 = acc4 * dinv4 + b1_4[None, :]
    hid = jnp.maximum(out1, 0.0)
    w24 = _blockdiag(w2_ref[...], p)
    hw2q = jnp.dot(hid, w24,
                   preferred_element_type=jnp.float32)
    hw2rep = jnp.dot(hw2q, _expand_mat(p, h),
                     preferred_element_type=jnp.float32)
    u2rep = hw2rep * dinv4
    u2_ref[...] = jnp.concatenate(
        [u2rep, jnp.zeros((q4 - n4, p * h), jnp.float32)], axis=0)

  return pl.pallas_call(
      body,
      out_shape=jax.ShapeDtypeStruct((q4, p * b1.shape[0]), jnp.float32),
  )(accp4, u4, dinvq, b1, w2)


def _tc_c(acc2lin, deglin, b2, nr):
  def body(acc2_ref, deg_ref, b2_ref, out_ref):
    deg = deg_ref[0] + deg_ref[1] + 1.0
    dinv = lax.rsqrt(deg)
    q = acc2_ref[0] + acc2_ref[1]
    out_ref[...] = jax.nn.sigmoid(q * dinv + b2_ref[0])

  return pl.pallas_call(
      body,
      out_shape=jax.ShapeDtypeStruct((nr, 128), jnp.float32),
  )(acc2lin, deglin, b2)


def kernel(x, edge_index, W1, b1, W2, b2):
  n = x.shape[0]
  d = x.shape[1]
  e = edge_index.shape[1]
  h = W1.shape[1]

  npad = ((n + NS * CHUNK - 1) // (NS * CHUNK)) * (NS * CHUNK)
  gsz = NW * CHUNK * K
  cpw = K * ((e + gsz - 1) // gsz)
  ep = NW * cpw * CHUNK

  npr = np.arange(ep - e, dtype=np.int32)
  pad2 = jnp.asarray(np.stack([npr % n, n + npr % (npad - n)]))
  slab = jnp.concatenate([edge_index, pad2], axis=1).reshape(
      2, NW, cpw, CHUNK)

  p = 128 // h
  n4 = n // p
  q4 = npad // p

  degp = _deg_kernel(npad, cpw)(slab)
  u4, dinvq = _tc_a(x.reshape(n4, p * d), W1,
                    degp.reshape(NC, q4, p), n4, q4, p)
  accp = _agg2d_kernel(npad, cpw, h)(u4.reshape(n, h), slab)
  u2rep = _tc_b(accp.reshape(NC, q4, 128), u4, dinvq, b1, W2, n4, q4, p)
  acc2p = _agg1d_kernel(npad, cpw)(u2rep, slab)
  outlin = _tc_c(acc2p.reshape(NC, npad // 128, 128),
                 degp.reshape(NC, npad // 128, 128), b2, npad // 128)
  return outlin.reshape(npad)[:n]
